# baseline (speedup 1.0000x reference)
import jax
import jax.numpy as jnp
from jax import lax
from jax.experimental import pallas as pl
from jax.experimental.pallas import tpu as pltpu
from jax.experimental.pallas import tpu_sc as plsc

QUANT_LEVELS = 1000
LATENT_DIM = 64
EMB_DIM = 6 * LATENT_DIM
B, L = 1024, 200
N = B * L

NC, NS = 2, 16
NW = NC * NS
B_PER_W = B // NW


def _sc_body(ids_hbm, table_hbm, x_hbm, idx_all, rows_v, gsem):
    wid = lax.axis_index("s") * NC + lax.axis_index("c")
    b0 = pl.multiple_of(wid * B_PER_W, B_PER_W)

    pltpu.sync_copy(ids_hbm.at[pl.ds(b0 * L, B_PER_W * L)], idx_all)

    def row_block(rb, carry):
        loc = pl.multiple_of(rb * L, 8)
        pltpu.async_copy(
            table_hbm.at[idx_all.at[pl.ds(loc, L)]], rows_v, gsem).wait()
        pltpu.sync_copy(rows_v, x_hbm.at[b0 + rb, :, :])
        return carry

    lax.fori_loop(0, B_PER_W, row_block, 0)


@jax.jit
def _sc_gather(ids_flat, table):
    scratch = [
        pltpu.VMEM((B_PER_W * L,), jnp.int32),
        pltpu.VMEM((L, EMB_DIM), jnp.float32),
        pltpu.SemaphoreType.DMA,
    ]
    mesh = plsc.VectorSubcoreMesh(core_axis_name="c", subcore_axis_name="s",
                                  num_cores=NC, num_subcores=NS)
    k = pl.kernel(_sc_body, out_type=jax.ShapeDtypeStruct((B, L, EMB_DIM),
                                                          jnp.float32),
                  mesh=mesh, scratch_types=scratch)
    return k(ids_flat, table)


def _softplus(v):
    return jnp.maximum(v, 0.0) + jnp.log1p(jnp.exp(-jnp.abs(v)))


def _sigmoid(v):
    return 1.0 / (1.0 + jnp.exp(-v))


_ACTS = (_softplus, _softplus, _sigmoid, _sigmoid, _sigmoid, _sigmoid)

_BB = 32


def _tc_body(x_ref, o0, o1, o2, o3, o4, o5):
    outs = (o0, o1, o2, o3, o4, o5)
    x = x_ref[...]
    for s in range(6):
        outs[s][...] = _ACTS[s](x[:, :, s * LATENT_DIM:(s + 1) * LATENT_DIM])


@jax.jit
def _tc_acts(x):
    d = LATENT_DIM
    out_shape = tuple(jax.ShapeDtypeStruct((B, L, d), jnp.float32)
                      for _ in range(6))
    return pl.pallas_call(
        _tc_body,
        grid=(B // _BB,),
        in_specs=[pl.BlockSpec((_BB, L, EMB_DIM), lambda i: (i, 0, 0))],
        out_specs=tuple(pl.BlockSpec((_BB, L, d), lambda i: (i, 0, 0))
                        for _ in range(6)),
        out_shape=out_shape,
    )(x)


def kernel(quant_ids, table):
    x = _sc_gather(quant_ids.reshape(N), table)
    return (x,) + tuple(_tc_acts(x))
